# baseline (speedup 1.0000x reference)
import functools

import jax
import jax.numpy as jnp
from jax import lax
from jax.experimental import pallas as pl
from jax.experimental.pallas import tpu as pltpu
from jax.experimental.pallas import tpu_sc as plsc

N = 100000
D = 128
S = 512
OUT = 128

NC = 2
NS = 16
NW = NC * NS
LANES = 16

CHUNK = 80
NCHUNKS = N // CHUNK


def _sc_pool(x_a, x_t, x_u, b_a, b_t, b_u):
    mesh = plsc.VectorSubcoreMesh(core_axis_name="c", subcore_axis_name="s")

    out_type = (
        [jax.ShapeDtypeStruct((NC, S, D), jnp.float32) for _ in range(3)]
        + [jax.ShapeDtypeStruct((NW, S), jnp.int32) for _ in range(3)]
    )

    nslots = 3
    NVR = CHUNK // LANES
    DUMMY = S
    ACC_ROWS = S + LANES
    CROWS = 2 * LANES
    scratch = dict(
        acc0=pltpu.VMEM_SHARED((ACC_ROWS, D), jnp.float32),
        acc1=pltpu.VMEM_SHARED((ACC_ROWS, D), jnp.float32),
        acc2=pltpu.VMEM_SHARED((ACC_ROWS, D), jnp.float32),
        bufs=[pltpu.VMEM((CHUNK, D), jnp.float32) for _ in range(nslots)],
        idxs=[pltpu.VMEM((CHUNK,), jnp.int32) for _ in range(nslots)],
        cbufs=[pltpu.VMEM((CROWS, D), jnp.float32) for _ in range(nslots)],
        sidxs=[pltpu.VMEM((CROWS,), jnp.int32) for _ in range(nslots)],
        zrow=pltpu.VMEM((ACC_ROWS // NS, D), jnp.float32),
        hist=pltpu.SMEM((S,), jnp.int32),
        histv=pltpu.VMEM((S,), jnp.int32),
        ld_i=[pltpu.SemaphoreType.DMA for _ in range(nslots)],
        ld_b=[pltpu.SemaphoreType.DMA for _ in range(nslots)],
        sc_d=[pltpu.SemaphoreType.DMA for _ in range(nslots)],
    )

    KMAIN = 39
    assert KMAIN % nslots == 0 and KMAIN * NW < NCHUNKS <= (KMAIN + 1) * NW

    @functools.partial(pl.kernel, out_type=out_type, mesh=mesh,
                       scratch_types=scratch)
    def k(xa, xt, xu, ba, bt, bu, sa, st, su, ca, ct, cu,
          acc0, acc1, acc2, bufs, idxs, cbufs, sidxs, zrow, hist, histv,
          ld_i, ld_b, sc_d):
        cid = lax.axis_index("c")
        sid = lax.axis_index("s")
        wid = sid * NC + cid
        rows_per = S // NS
        iota = lax.iota(jnp.int32, LANES)
        zvec = jnp.zeros((LANES,), jnp.float32)

        def _zero_row(r, carry):
            for kk in range(D // LANES):
                zrow[r, pl.ds(kk * LANES, LANES)] = zvec
            return carry
        lax.fori_loop(0, ACC_ROWS // NS, _zero_row, 0)

        zr0 = sid * (ACC_ROWS // NS)
        for acc in (acc0, acc1, acc2):
            pltpu.sync_copy(zrow, acc.at[pl.ds(zr0, ACC_ROWS // NS)])
        plsc.subcore_barrier()

        def _start_load(x_hbm, b_hbm, kc, j):
            base = (wid + kc * NW) * CHUNK
            pltpu.async_copy(b_hbm.at[pl.ds(base, CHUNK)], idxs[j], ld_i[j])
            pltpu.async_copy(x_hbm.at[pl.ds(base, CHUNK)], bufs[j], ld_b[j])

        def _wait_load(x_hbm, b_hbm, j):
            pltpu.make_async_copy(b_hbm.at[pl.ds(0, CHUNK)], idxs[j], ld_i[j]).wait()
            pltpu.make_async_copy(x_hbm.at[pl.ds(0, CHUNK)], bufs[j], ld_b[j]).wait()

        def _chunk_compact(j, acc):
            buf, idx, cbuf, sidx = bufs[j], idxs[j], cbufs[j], sidxs[j]
            sumids = jnp.full((LANES,), DUMMY, jnp.int32)
            nheavy = jnp.int32(0)
            hg = jnp.int32(0)
            for g in range(NVR):
                v = idx[pl.ds(g * LANES, LANES)]
                f = v[0]
                l = v[LANES - 1]
                light = f == l

                def _gsum(t, acc8, g=g):
                    return tuple(a + buf[g * LANES + t, pl.ds(kk * LANES, LANES)]
                                 for kk, a in enumerate(acc8))
                acc8 = lax.fori_loop(0, LANES, _gsum,
                                     tuple([zvec] * (D // LANES)))
                for kk, a in enumerate(acc8):
                    cbuf[g, pl.ds(kk * LANES, LANES)] = a
                sumids = jnp.where(iota == g,
                                   jnp.where(light, f, DUMMY), sumids)
                heavy_i = 1 - light.astype(jnp.int32)
                nheavy = nheavy + heavy_i
                hg = hg + g * heavy_i

                @pl.when(light)
                def _hl(f=f):
                    hist[f] = hist[f] + LANES

                @pl.when(jnp.logical_not(light))
                def _hh(v=v):
                    for lane in range(LANES):
                        s = v[lane]
                        hist[s] = hist[s] + 1

            sidx[pl.ds(0, LANES)] = sumids

            @pl.when(nheavy == 1)
            def _inline_heavy():
                sidx[pl.ds(LANES, LANES)] = idx[pl.ds(hg * LANES, LANES)]

                def _cp(t, carry):
                    for kk in range(D // LANES):
                        cbuf[LANES + t, pl.ds(kk * LANES, LANES)] = (
                            buf[hg * LANES + t, pl.ds(kk * LANES, LANES)])
                    return carry
                lax.fori_loop(0, LANES, _cp, 0)

            @pl.when(nheavy != 1)
            def _no_heavy():
                sidx[pl.ds(LANES, LANES)] = jnp.full((LANES,), DUMMY, jnp.int32)

            @pl.when(nheavy >= 2)
            def _fallback():
                sidx[pl.ds(0, LANES)] = jnp.full((LANES,), DUMMY, jnp.int32)
                pltpu.sync_copy(buf, acc.at[idx], add=True)

        def _process(x_hbm, b_hbm, acc, cout, phase):
            def _zh(i, carry):
                hist[i] = jnp.int32(0)
                return carry
            lax.fori_loop(0, S, _zh, 0)

            for j in range(2):
                _start_load(x_hbm, b_hbm, j, j)

            def _body(ip, carry):
                for j in range(nslots):
                    kc = ip * nslots + j
                    _wait_load(x_hbm, b_hbm, j)

                    @pl.when(kc >= nslots)
                    def _drain(j=j):
                        pltpu.make_async_copy(cbufs[j], acc.at[sidxs[j]],
                                              sc_d[j]).wait()
                    knext = jnp.minimum(kc + 2, KMAIN - 1)
                    _start_load(x_hbm, b_hbm, knext, (j + 2) % nslots)
                    _chunk_compact(j, acc)
                    pltpu.async_copy(cbufs[j], acc.at[sidxs[j]], sc_d[j], add=True)
                return carry
            lax.fori_loop(0, KMAIN // nslots, _body, 0)

            for j in range(nslots):
                pltpu.make_async_copy(cbufs[j], acc.at[sidxs[j]], sc_d[j]).wait()
            for kc in (KMAIN - 2, KMAIN - 1):
                jdup = ((kc % nslots) + 2) % nslots
                _wait_load(x_hbm, b_hbm, jdup)

            ntail = NCHUNKS - KMAIN * NW
            tail_rank = wid - phase * ntail
            @pl.when(jnp.logical_and(tail_rank >= 0, tail_rank < ntail))
            def _tail():
                base = (KMAIN * NW + tail_rank) * CHUNK
                pltpu.sync_copy(b_hbm.at[pl.ds(base, CHUNK)], idxs[0])
                pltpu.sync_copy(x_hbm.at[pl.ds(base, CHUNK)], bufs[0])
                _chunk_compact(0, acc)
                pltpu.sync_copy(cbufs[0], acc.at[sidxs[0]], add=True)

            def _bg(g, carry):
                vec = jnp.zeros((LANES,), jnp.int32)
                for lane in range(LANES):
                    s = hist[g * LANES + lane]
                    vec = jnp.where(iota == lane, s, vec)
                histv[pl.ds(g * LANES, LANES)] = vec
                return carry
            lax.fori_loop(0, S // LANES, _bg, 0)
            pltpu.sync_copy(histv, cout.at[wid])

        _process(xa, ba, acc0, ca, 0)
        _process(xt, bt, acc1, ct, 1)
        _process(xu, bu, acc2, cu, 2)
        plsc.subcore_barrier()

        row0 = sid * rows_per
        for acc, out in ((acc0, sa), (acc1, st), (acc2, su)):
            pltpu.sync_copy(acc.at[pl.ds(row0, rows_per)],
                            out.at[cid, pl.ds(row0, rows_per)])

    return k(x_a, x_t, x_u, b_a, b_t, b_u)


def _tc_finish_body(sa, st, su, ca, ct, cu, w_ref, b_ref, out_ref):
    w = w_ref[...]
    out = jnp.broadcast_to(b_ref[...], (S, OUT))
    for a, (s_ref, c_ref) in enumerate(((sa, ca), (st, ct), (su, cu))):
        tot = s_ref[0] + s_ref[1]
        cnt = jnp.sum(c_ref[...], axis=0).astype(jnp.float32)[:, None]
        mean = tot / jnp.maximum(cnt, 1.0)
        out = out + lax.dot_general(
            mean, w[:, a * D:(a + 1) * D],
            dimension_numbers=(((1,), (1,)), ((), ())),
            preferred_element_type=jnp.float32,
            precision=lax.Precision.HIGHEST,
        )
    out_ref[...] = out


def kernel(x_article, x_tweet, x_user, batch_article, batch_tweet, batch_user, W, b):
    sa, st, su, ca, ct, cu = _sc_pool(
        x_article, x_tweet, x_user, batch_article, batch_tweet, batch_user)
    return pl.pallas_call(
        _tc_finish_body,
        out_shape=jax.ShapeDtypeStruct((S, OUT), jnp.float32),
    )(sa, st, su, ca, ct, cu, W, b.reshape(1, OUT))


# speedup vs baseline: 1.3151x; 1.3151x over previous
import functools

import jax
import jax.numpy as jnp
from jax import lax
from jax.experimental import pallas as pl
from jax.experimental.pallas import tpu as pltpu
from jax.experimental.pallas import tpu_sc as plsc

N = 100000
D = 128
S = 512
OUT = 128

NC = 2
NS = 16
NW = NC * NS
LANES = 16

CHUNK = 80
NCHUNKS = N // CHUNK


def _sc_pool(x_a, x_t, x_u, b_a, b_t, b_u):
    mesh = plsc.VectorSubcoreMesh(core_axis_name="c", subcore_axis_name="s")

    out_type = (
        [jax.ShapeDtypeStruct((NC, S, D), jnp.float32) for _ in range(3)]
        + [jax.ShapeDtypeStruct((NW, S), jnp.int32) for _ in range(3)]
    )

    nslots = 3
    NVR = CHUNK // LANES
    scratch = dict(
        acc0=pltpu.VMEM_SHARED((S, D), jnp.float32),
        acc1=pltpu.VMEM_SHARED((S, D), jnp.float32),
        acc2=pltpu.VMEM_SHARED((S, D), jnp.float32),
        bufs=[pltpu.VMEM((CHUNK, D), jnp.float32) for _ in range(nslots)],
        idxs=[pltpu.VMEM((CHUNK,), jnp.int32) for _ in range(nslots)],
        zrow=pltpu.VMEM((S // NS, D), jnp.float32),
        hist=pltpu.SMEM((S,), jnp.int32),
        histv=pltpu.VMEM((S,), jnp.int32),
        ld_i=[pltpu.SemaphoreType.DMA for _ in range(nslots)],
        ld_b=[pltpu.SemaphoreType.DMA for _ in range(nslots)],
        sc_d=[pltpu.SemaphoreType.DMA for _ in range(nslots)],
    )

    KMAIN = 39
    assert KMAIN % nslots == 0 and KMAIN * NW < NCHUNKS <= (KMAIN + 1) * NW

    @functools.partial(pl.kernel, out_type=out_type, mesh=mesh,
                       scratch_types=scratch)
    def k(xa, xt, xu, ba, bt, bu, sa, st, su, ca, ct, cu,
          acc0, acc1, acc2, bufs, idxs, zrow, hist, histv,
          ld_i, ld_b, sc_d):
        cid = lax.axis_index("c")
        sid = lax.axis_index("s")
        wid = sid * NC + cid
        rows_per = S // NS
        iota = lax.iota(jnp.int32, LANES)

        def _zero_row(r, carry):
            for kk in range(D // LANES):
                zrow[r, pl.ds(kk * LANES, LANES)] = jnp.zeros((LANES,), jnp.float32)
            return carry
        lax.fori_loop(0, rows_per, _zero_row, 0)

        row0 = sid * rows_per
        for acc in (acc0, acc1, acc2):
            pltpu.sync_copy(zrow, acc.at[pl.ds(row0, rows_per)])
        plsc.subcore_barrier()

        def _start_load(x_hbm, b_hbm, kc, j):
            base = (wid + kc * NW) * CHUNK
            pltpu.async_copy(b_hbm.at[pl.ds(base, CHUNK)], idxs[j], ld_i[j])
            pltpu.async_copy(x_hbm.at[pl.ds(base, CHUNK)], bufs[j], ld_b[j])

        def _wait_load(x_hbm, b_hbm, j):
            pltpu.make_async_copy(b_hbm.at[pl.ds(0, CHUNK)], idxs[j], ld_i[j]).wait()
            pltpu.make_async_copy(x_hbm.at[pl.ds(0, CHUNK)], bufs[j], ld_b[j]).wait()

        def _hist_update(j):
            for kk in range(NVR):
                v = idxs[j][pl.ds(kk * LANES, LANES)]
                f = v[0]
                l = v[LANES - 1]

                @pl.when(f == l)
                def _light(f=f):
                    hist[f] = hist[f] + LANES

                @pl.when(f != l)
                def _heavy(v=v):
                    for lane in range(LANES):
                        s = v[lane]
                        hist[s] = hist[s] + 1

        def _process(x_hbm, b_hbm, acc, cout, phase):
            def _zh(i, carry):
                hist[i] = jnp.int32(0)
                return carry
            lax.fori_loop(0, S, _zh, 0)

            for j in range(2):
                _start_load(x_hbm, b_hbm, j, j)

            def _body(ip, carry):
                for j in range(nslots):
                    kc = ip * nslots + j
                    _wait_load(x_hbm, b_hbm, j)
                    pltpu.async_copy(bufs[j], acc.at[idxs[j]], sc_d[j], add=True)
                    _hist_update(j)
                    jprev = (j + 2) % nslots

                    @pl.when(kc > 0)
                    def _drain(jprev=jprev):
                        pltpu.make_async_copy(bufs[jprev], acc.at[idxs[jprev]],
                                              sc_d[jprev]).wait()
                    knext = jnp.minimum(kc + 2, KMAIN - 1)
                    _start_load(x_hbm, b_hbm, knext, jprev)
                return carry
            lax.fori_loop(0, KMAIN // nslots, _body, 0)

            jlast = (KMAIN - 1) % nslots
            pltpu.make_async_copy(bufs[jlast], acc.at[idxs[jlast]],
                                  sc_d[jlast]).wait()
            for kc in (KMAIN - 2, KMAIN - 1):
                jdup = ((kc % nslots) + 2) % nslots
                _wait_load(x_hbm, b_hbm, jdup)

            ntail = NCHUNKS - KMAIN * NW
            tail_rank = wid - phase * ntail
            @pl.when(jnp.logical_and(tail_rank >= 0, tail_rank < ntail))
            def _tail():
                base = (KMAIN * NW + tail_rank) * CHUNK
                pltpu.sync_copy(b_hbm.at[pl.ds(base, CHUNK)], idxs[0])
                pltpu.sync_copy(x_hbm.at[pl.ds(base, CHUNK)], bufs[0])
                pltpu.sync_copy(bufs[0], acc.at[idxs[0]], add=True)
                _hist_update(0)

            def _bg(g, carry):
                vec = jnp.zeros((LANES,), jnp.int32)
                for lane in range(LANES):
                    s = hist[g * LANES + lane]
                    vec = jnp.where(iota == lane, s, vec)
                histv[pl.ds(g * LANES, LANES)] = vec
                return carry
            lax.fori_loop(0, S // LANES, _bg, 0)
            pltpu.sync_copy(histv, cout.at[wid])

        _process(xa, ba, acc0, ca, 0)
        _process(xt, bt, acc1, ct, 1)
        _process(xu, bu, acc2, cu, 2)
        plsc.subcore_barrier()

        for acc, out in ((acc0, sa), (acc1, st), (acc2, su)):
            pltpu.sync_copy(acc.at[pl.ds(row0, rows_per)],
                            out.at[cid, pl.ds(row0, rows_per)])

    return k(x_a, x_t, x_u, b_a, b_t, b_u)


def _tc_finish_body(sa, st, su, ca, ct, cu, w_ref, b_ref, out_ref):
    w = w_ref[...]
    out = jnp.broadcast_to(b_ref[...], (S, OUT))
    for a, (s_ref, c_ref) in enumerate(((sa, ca), (st, ct), (su, cu))):
        tot = s_ref[0] + s_ref[1]
        cnt = jnp.sum(c_ref[...], axis=0).astype(jnp.float32)[:, None]
        mean = tot / jnp.maximum(cnt, 1.0)
        out = out + lax.dot_general(
            mean, w[:, a * D:(a + 1) * D],
            dimension_numbers=(((1,), (1,)), ((), ())),
            preferred_element_type=jnp.float32,
            precision=lax.Precision.HIGHEST,
        )
    out_ref[...] = out


def kernel(x_article, x_tweet, x_user, batch_article, batch_tweet, batch_user, W, b):
    sa, st, su, ca, ct, cu = _sc_pool(
        x_article, x_tweet, x_user, batch_article, batch_tweet, batch_user)
    return pl.pallas_call(
        _tc_finish_body,
        out_shape=jax.ShapeDtypeStruct((S, OUT), jnp.float32),
    )(sa, st, su, ca, ct, cu, W, b.reshape(1, OUT))
